# baseline (speedup 1.0000x reference)
import functools

import jax
import jax.numpy as jnp
from jax import lax
from jax.experimental import pallas as pl
from jax.experimental.pallas import tpu as pltpu
from jax.experimental.pallas import tpu_sc as plsc

F32 = jnp.float32
BF16 = jnp.bfloat16

_NC = 2
_NS = 16
_NW = _NC * _NS

_BN = 2000
_BE = 1280
_CH = 200

_SPLIT = (81920, 78080)


def _dot(a, b):
    return lax.dot_general(a, b, (((1,), (0,)), ((), ())),
                           precision=lax.Precision.DEFAULT,
                           preferred_element_type=F32)


def _ln(h, g, b):
    d = h.shape[-1]
    s1 = jnp.sum(h, axis=-1, keepdims=True)
    s2 = jnp.sum(h * h, axis=-1, keepdims=True)
    mu = s1 * (1.0 / d)
    var = s2 * (1.0 / d) - mu * mu
    return (h - mu) * lax.rsqrt(var + 1e-5) * g + b


def _mlp2_ln(pre1, w2, b2, g, b):
    h = jnp.maximum(pre1, 0.0)
    h = jnp.maximum(_dot(h, w2) + b2, 0.0)
    return _ln(h, g, b)


def _row(v):
    return v.reshape(1, -1)


def _mlp_parts(p):
    (l1, l2) = p['layers']
    return (l1['w'], _row(l1['b']), l2['w'], _row(l2['b']),
            _row(p['norm']['g']), _row(p['norm']['b']))


def _full(shape):
    return pl.BlockSpec(shape, lambda i: tuple(0 for _ in shape))


def _rows(b, d, off=0):
    return pl.BlockSpec((b, d), lambda i: (i + off, 0))


def _pc_encode_edge(ea, p):
    e, din = ea.shape
    w1, b1, w2, b2, g, b = _mlp_parts(p)

    def kfn(ea_r, w1_r, b1_r, w2_r, b2_r, g_r, b_r, o_r):
        pre = _dot(ea_r[...], w1_r[...]) + b1_r[...]
        o_r[...] = _mlp2_ln(pre, w2_r[...], b2_r[...], g_r[...],
                            b_r[...]).astype(BF16)

    return pl.pallas_call(
        kfn,
        grid=(e // _BE,),
        in_specs=[_rows(_BE, din), _full(w1.shape), _full(b1.shape),
                  _full(w2.shape), _full(b2.shape), _full(g.shape), _full(b.shape)],
        out_specs=_rows(_BE, 128),
        out_shape=jax.ShapeDtypeStruct((e, 128), BF16),
    )(ea, w1, b1, w2, b2, g, b)


def _pc_encode_node(x, u, pn, pg, w1n, b1n, v1n, c1n):
    n, din = x.shape
    w1, b1, w2, b2, g, b = _mlp_parts(pn)
    uw1, ub1, uw2, ub2, ug, ub = _mlp_parts(pg)

    def kfn(x_r, w1_r, b1_r, w2_r, b2_r, g_r, b_r,
            u_r, uw1_r, ub1_r, uw2_r, ub2_r, ug_r, ub_r,
            w1n_r, b1n_r, v1n_r, c1n_r,
            n_o, g_o, a_o, b_o, ce_o, cn_o):
        pre = _dot(x_r[...], w1_r[...]) + b1_r[...]
        n0 = _mlp2_ln(pre, w2_r[...], b2_r[...], g_r[...], b_r[...])
        n_o[...] = n0
        a_o[...] = _dot(n0, w1n_r[256:384, :] + w1n_r[384:512, :])
        b_o[...] = _dot(n0, w1n_r[512:640, :] + w1n_r[640:768, :])

        @pl.when(pl.program_id(0) == 0)
        def _():
            gp = _dot(u_r[...], uw1_r[...]) + ub1_r[...]
            g0 = _mlp2_ln(gp, uw2_r[...], ub2_r[...], ug_r[...], ub_r[...])
            g_o[...] = g0
            ce_o[...] = _dot(g0, w1n_r[768:896, :] + w1n_r[896:1024, :]) + b1n_r[...]
            cn_o[...] = _dot(g0, v1n_r[384:512, :] + v1n_r[512:640, :]) + c1n_r[...]

    return pl.pallas_call(
        kfn,
        grid=(n // _BN,),
        in_specs=[_rows(_BN, din)] + [_full(a.shape) for a in
                                      (w1, b1, w2, b2, g, b,
                                       u, uw1, ub1, uw2, ub2, ug, ub,
                                       w1n, b1n, v1n, c1n)],
        out_specs=(_rows(_BN, 128), _full((1, 128)),
                   _rows(_BN, 128), _rows(_BN, 128),
                   _full((1, 128)), _full((1, 128))),
        out_shape=tuple(jax.ShapeDtypeStruct(s, F32)
                        for s in ((n, 128), (1, 128), (n, 128), (n, 128),
                                  (1, 128), (1, 128))),
    )(x, w1, b1, w2, b2, g, b, u, uw1, ub1, uw2, ub2, ug, ub, w1n, b1n, v1n, c1n)


def _sc_gather(table_a, table_b, src_h, dst_h):
    e = src_h.shape[0]
    per_w = e // _NW
    n_full = per_w // _CH
    tail = per_w % _CH
    mesh = plsc.VectorSubcoreMesh(core_axis_name="c", subcore_axis_name="s")

    scratch = [pltpu.VMEM((_CH,), jnp.int32), pltpu.VMEM((_CH,), jnp.int32),
               pltpu.VMEM((_CH, 128), F32), pltpu.VMEM((_CH, 128), F32),
               pltpu.SemaphoreType.DMA, pltpu.SemaphoreType.DMA]
    if tail:
        scratch += [pltpu.VMEM((tail,), jnp.int32), pltpu.VMEM((tail,), jnp.int32),
                    pltpu.VMEM((tail, 128), F32), pltpu.VMEM((tail, 128), F32)]

    @functools.partial(
        pl.kernel,
        out_type=(jax.ShapeDtypeStruct((e, 128), F32),
                  jax.ShapeDtypeStruct((e, 128), F32)),
        mesh=mesh,
        scratch_types=scratch,
    )
    def k(ta, tb, s_h, d_h, ga, gb, ia, ib, ra, rb, sa, sb, *tails):
        wid = lax.axis_index("s") * _NC + lax.axis_index("c")
        base = wid * per_w

        def chunk(off, cia, cib, cra, crb, cn):
            pltpu.sync_copy(s_h.at[pl.ds(off, cn)], cia)
            pltpu.sync_copy(d_h.at[pl.ds(off, cn)], cib)
            cpa = pltpu.async_copy(ta.at[cia], cra, sa)
            cpb = pltpu.async_copy(tb.at[cib], crb, sb)
            cpa.wait()
            cpb.wait()
            pltpu.sync_copy(cra, ga.at[pl.ds(off, cn)])
            pltpu.sync_copy(crb, gb.at[pl.ds(off, cn)])

        def body(ci, carry):
            chunk(base + ci * _CH, ia, ib, ra, rb, _CH)
            return carry

        lax.fori_loop(0, n_full, body, 0)
        if tail:
            ia_t, ib_t, ra_t, rb_t = tails
            chunk(base + n_full * _CH, ia_t, ib_t, ra_t, rb_t, tail)

    return k(table_a, table_b, src_h, dst_h)


def _sc_scatter(e2_h, dst_h, zeros_nd):
    e = dst_h.shape[0]
    n = zeros_nd.shape[0]
    per_w = e // _NW
    n_full = per_w // _CH
    tail = per_w % _CH
    rpt = (n // _NS) // 8 * 8
    rtail = n - _NS * rpt
    mesh = plsc.VectorSubcoreMesh(core_axis_name="c", subcore_axis_name="s")

    scratch = [pltpu.VMEM((_CH,), jnp.int32), pltpu.VMEM((_CH, 128), F32),
               pltpu.VMEM_SHARED((n, 128), F32)]
    if tail:
        scratch += [pltpu.VMEM((tail,), jnp.int32), pltpu.VMEM((tail, 128), F32)]

    @functools.partial(
        pl.kernel,
        out_type=jax.ShapeDtypeStruct((2, n, 128), F32),
        mesh=mesh,
        scratch_types=scratch,
    )
    def k(e2h, d_h, z_h, out_h, idx, buf, acc, *tails):
        cid = lax.axis_index("c")
        sid = lax.axis_index("s")
        wid = sid * _NC + cid

        @pl.when(sid == 0)
        def _():
            pltpu.sync_copy(z_h, acc)

        plsc.subcore_barrier()
        base = wid * per_w

        def chunk(off, cidx, cbuf, cn):
            pltpu.sync_copy(d_h.at[pl.ds(off, cn)], cidx)
            pltpu.sync_copy(e2h.at[pl.ds(off, cn)], cbuf)
            pltpu.sync_copy(cbuf, acc.at[cidx], add=True)

        def body(ci, carry):
            chunk(base + ci * _CH, idx, buf, _CH)
            return carry

        lax.fori_loop(0, n_full, body, 0)
        if tail:
            idx_t, buf_t = tails
            chunk(base + n_full * _CH, idx_t, buf_t, tail)
        plsc.subcore_barrier()
        r0 = sid * rpt
        pltpu.sync_copy(acc.at[pl.ds(r0, rpt)], out_h.at[cid, pl.ds(r0, rpt)])
        if rtail:
            @pl.when(sid == _NS - 1)
            def _():
                t0 = _NS * rpt
                pltpu.sync_copy(acc.at[pl.ds(t0, rtail)],
                                out_h.at[cid, pl.ds(t0, rtail)])

    return k(e2_h, dst_h, zeros_nd)


def _pc_edge_mlp(e0, ec, ga, gb, w1, ce, w2, b2, g, b, want_sum, off_b, first):
    e = ga.shape[0]

    def kfn(*refs):
        if first:
            (e0_r, ga_r, gb_r, w1_r, ce_r, w2_r, b2_r, g_r, b_r) = refs[:9]
            outs = refs[9:]
            pre = (_dot(e0_r[...].astype(F32), w1_r[0:128, :] + w1_r[128:256, :]) +
                   ga_r[...] + gb_r[...] + ce_r[...])
        else:
            (e0_r, ec_r, ga_r, gb_r, w1_r, ce_r, w2_r, b2_r, g_r, b_r) = refs[:10]
            outs = refs[10:]
            pre = (_dot(e0_r[...].astype(F32), w1_r[0:128, :]) +
                   _dot(ec_r[...], w1_r[128:256, :]) +
                   ga_r[...] + gb_r[...] + ce_r[...])
        e2 = _mlp2_ln(pre, w2_r[...], b2_r[...], g_r[...], b_r[...])
        outs[0][...] = e2
        if want_sum:
            s = jnp.sum(e2, axis=0, keepdims=True)

            @pl.when(pl.program_id(0) == 0)
            def _():
                outs[1][...] = s

            @pl.when(pl.program_id(0) != 0)
            def _():
                outs[1][...] += s

    ins = ([e0] if first else [e0, ec]) + [ga, gb, w1, ce, w2, b2, g, b]
    in_specs = [_rows(_BE, 128, off=off_b)] + \
               [_rows(_BE, 128)] * (2 if first else 3) + \
               [_full(a.shape) for a in ins[(3 if first else 4):]]
    out_specs = (_rows(_BE, 128),) + ((_full((1, 128)),) if want_sum else ())
    out_shape = ((jax.ShapeDtypeStruct((e, 128), F32),) +
                 ((jax.ShapeDtypeStruct((1, 128), F32),) if want_sum else ()))
    r = pl.pallas_call(
        kfn, grid=(e // _BE,), in_specs=in_specs,
        out_specs=out_specs if want_sum else out_specs[0],
        out_shape=out_shape if want_sum else out_shape[0],
    )(*ins)
    return r if want_sum else (r, None)


def _pc_node_mlp(n0, nc, parts, cn, v1, v2, c2, g, b,
                 glob_args, table_args, first, e_count):
    n = n0.shape[0]
    do_glob = glob_args is not None
    do_tab = table_args is not None
    nb = n // _BN
    np_ = len(parts)

    if do_glob:
        (g0, gc, esums, g1, cg1, gw2, cg2, ggm, gbt) = glob_args
    if do_tab:
        (w1n, b1n, v1n, c1n) = table_args

    def kfn(*refs):
        i = 0
        n0_r = refs[i]; i += 1
        if not first:
            nc_r = refs[i]; i += 1
        p_rs = refs[i:i + np_]; i += np_
        cn_r, v1_r, v2_r, c2_r, g_r, b_r = refs[i:i + 6]; i += 6
        if do_glob:
            es_rs = refs[i:i + len(esums)]; i += len(esums)
            (g0_r, gc_r, g1_r, cg1_r, gw2_r, cg2_r, ggm_r, gbt_r) = refs[i:i + 8]
            i += 8
        if do_tab:
            (w1n_r, b1n_r, v1n_r, c1n_r) = refs[i:i + 4]; i += 4
        outs = refs[i:]

        agg = p_rs[0][0] + p_rs[0][1]
        for p_r in p_rs[1:]:
            agg = agg + p_r[0] + p_r[1]
        if first:
            pre = (_dot(n0_r[...], v1_r[0:128, :] + v1_r[128:256, :]) +
                   _dot(agg, v1_r[256:384, :]) + cn_r[...])
        else:
            pre = (_dot(n0_r[...], v1_r[0:128, :]) +
                   _dot(nc_r[...], v1_r[128:256, :]) +
                   _dot(agg, v1_r[256:384, :]) + cn_r[...])
        n2 = _mlp2_ln(pre, v2_r[...], c2_r[...], g_r[...], b_r[...])
        outs[0][...] = n2
        oi = 1
        if do_tab:
            outs[oi][...] = _dot(n0_r[...], w1n_r[256:384, :]) + \
                _dot(n2, w1n_r[384:512, :])
            outs[oi + 1][...] = _dot(n0_r[...], w1n_r[512:640, :]) + \
                _dot(n2, w1n_r[640:768, :])
            oi += 2
        if do_glob:
            s = jnp.sum(n2, axis=0, keepdims=True)

            @pl.when(pl.program_id(0) == 0)
            def _():
                outs[oi][...] = s

            @pl.when(pl.program_id(0) != 0)
            def _():
                outs[oi][...] += s

            @pl.when(pl.program_id(0) == nb - 1)
            def _():
                nm = outs[oi][...] * (1.0 / n)
                em = es_rs[0][...]
                for es_r in es_rs[1:]:
                    em = em + es_r[...]
                em = em * (1.0 / e_count)
                if first:
                    gpre = _dot(g0_r[...], g1_r[0:128, :] + g1_r[128:256, :])
                else:
                    gpre = (_dot(g0_r[...], g1_r[0:128, :]) +
                            _dot(gc_r[...], g1_r[128:256, :]))
                gpre += (_dot(nm, g1_r[256:384, :]) +
                         _dot(em, g1_r[384:512, :]) + cg1_r[...])
                g2 = _mlp2_ln(gpre, gw2_r[...], cg2_r[...], ggm_r[...], gbt_r[...])
                outs[oi + 1][...] = g2
                if do_tab:
                    outs[oi + 2][...] = (_dot(g0_r[...], w1n_r[768:896, :]) +
                                         _dot(g2, w1n_r[896:1024, :]) + b1n_r[...])
                    outs[oi + 3][...] = (_dot(g0_r[...], v1n_r[384:512, :]) +
                                         _dot(g2, v1n_r[512:640, :]) + c1n_r[...])

    ins = [n0] + ([] if first else [nc]) + list(parts) + [cn, v1, v2, c2, g, b]
    if do_glob:
        ins += list(esums) + [g0, gc, g1, cg1, gw2, cg2, ggm, gbt]
    if do_tab:
        ins += [w1n, b1n, v1n, c1n]
    n_row = 1 + (0 if first else 1)
    in_specs = ([_rows(_BN, 128)] * n_row +
                [pl.BlockSpec((2, _BN, 128), lambda i: (0, i, 0))] * np_ +
                [_full(a.shape) for a in ins[n_row + np_:]])

    out_specs = [_rows(_BN, 128)]
    out_shape = [jax.ShapeDtypeStruct((n, 128), F32)]
    if do_tab:
        out_specs += [_rows(_BN, 128), _rows(_BN, 128)]
        out_shape += [jax.ShapeDtypeStruct((n, 128), F32)] * 2
    if do_glob:
        out_specs += [_full((1, 128)), _full((1, 128))]
        out_shape += [jax.ShapeDtypeStruct((1, 128), F32)] * 2
        if do_tab:
            out_specs += [_full((1, 128)), _full((1, 128))]
            out_shape += [jax.ShapeDtypeStruct((1, 128), F32)] * 2
    r = pl.pallas_call(
        kfn, grid=(nb,), in_specs=in_specs,
        out_specs=tuple(out_specs), out_shape=tuple(out_shape),
    )(*ins)
    return r


def _pc_node_decode(n0, nc, parts, cn, v1, v2, c2, g, b, y, pd, po):
    n = n0.shape[0]
    nb = n // _BN
    np_ = len(parts)
    dw1, db1, dw2, db2, dg, dbt = _mlp_parts(pd)
    ow = po['layers'][0]['w']
    ob = po['layers'][0]['b']
    owp = jnp.zeros((128, 128), F32).at[:, :ow.shape[1]].set(ow)
    obp = jnp.zeros((1, 128), F32).at[0, :ob.shape[0]].set(ob)
    ny = y.shape[0]

    def kfn(*refs):
        (n0_r, nc_r) = refs[:2]
        p_rs = refs[2:2 + np_]
        (cn_r, v1_r, v2_r, c2_r, g_r, b_r,
         y_r, dw1_r, db1_r, dw2_r, db2_r, dg_r, dbt_r,
         owp_r, obp_r, acc) = refs[2 + np_:]

        agg = p_rs[0][0] + p_rs[0][1]
        for p_r in p_rs[1:]:
            agg = agg + p_r[0] + p_r[1]
        pre = (_dot(n0_r[...], v1_r[0:128, :]) +
               _dot(nc_r[...], v1_r[128:256, :]) +
               _dot(agg, v1_r[256:384, :]) + cn_r[...])
        n2 = _mlp2_ln(pre, v2_r[...], c2_r[...], g_r[...], b_r[...])

        base = _dot(n2, dw1_r[0:128, :]) + db1_r[...]
        yc = _dot(y_r[...], dw1_r[128:, :])

        @pl.when(pl.program_id(0) == 0)
        def _():
            acc[...] = jnp.full((ny, 128), jnp.inf, F32)

        for i in range(ny):
            h = _mlp2_ln(base + yc[i:i + 1, :], dw2_r[...], db2_r[...],
                         dg_r[...], dbt_r[...])
            o = _dot(h, owp_r[...]) + obp_r[...]
            m = jnp.min(o, axis=0, keepdims=True)
            acc[i:i + 1, :] = jnp.minimum(acc[i:i + 1, :], m)

    ins = [n0, nc] + list(parts) + [cn, v1, v2, c2, g, b,
                                    y, dw1, db1, dw2, db2, dg, dbt, owp, obp]
    in_specs = ([_rows(_BN, 128)] * 2 +
                [pl.BlockSpec((2, _BN, 128), lambda i: (0, i, 0))] * np_ +
                [_full(a.shape) for a in ins[2 + np_:]])
    acc = pl.pallas_call(
        kfn, grid=(nb,), in_specs=in_specs,
        out_specs=_full((ny, 128)),
        out_shape=jax.ShapeDtypeStruct((ny, 128), F32),
    )(*ins)
    return acc[:, :ow.shape[1]].reshape(-1)


def kernel(edge_attr, edge_index, x, y, z, u, batch, params):
    del z, batch
    e_count = edge_attr.shape[0]
    n_count = x.shape[0]

    if e_count == sum(_SPLIT):
        sizes = _SPLIT
    else:
        sizes = (e_count,)
    bounds = []
    o = 0
    for sz in sizes:
        bounds.append((o, sz))
        o += sz

    src_h = [lax.slice_in_dim(edge_index[0], s, s + sz) for s, sz in bounds]
    dst_h = [lax.slice_in_dim(edge_index[1], s, s + sz) for s, sz in bounds]

    enc = params['encoder']
    procs = params['processors']
    e0 = _pc_encode_edge(edge_attr, enc['edge'])

    w1_l, b1_l, w2_l, b2_l, egm_l, ebt_l = zip(*[_mlp_parts(p['edge'])
                                                 for p in procs])
    v1_l, c1_l, v2_l, c2_l, ngm_l, nbt_l = zip(*[_mlp_parts(p['node'])
                                                 for p in procs])

    n0, g0, tA, tB, ce, cn = _pc_encode_node(
        x, u, enc['node'], enc['glob'], w1_l[0], b1_l[0], v1_l[0], c1_l[0])

    zeros_nd = jnp.zeros((n_count, 128), F32)
    ec_h = [None] * len(bounds)
    n_cur = g_cur = None
    for i in range(3):
        first = i == 0
        last = i == 2

        gab_h = [_sc_gather(tA, tB, s, d) for s, d in zip(src_h, dst_h)]

        e_new_h = []
        esums = []
        parts = []
        for hi, (off, sz) in enumerate(bounds):
            ga, gb = gab_h[hi]
            e_new, esum = _pc_edge_mlp(
                e0, ec_h[hi], ga, gb, w1_l[i], ce, w2_l[i], b2_l[i],
                egm_l[i], ebt_l[i], want_sum=not last, off_b=off // _BE,
                first=first)
            e_new_h.append(e_new)
            if esum is not None:
                esums.append(esum)
            parts.append(_sc_scatter(e_new, dst_h[hi], zeros_nd))

        if last:
            pred = _pc_node_decode(n0, n_cur, parts, cn, v1_l[i], v2_l[i],
                                   c2_l[i], ngm_l[i], nbt_l[i], y,
                                   params['decoder']['node'],
                                   params['output_transformer']['node'])
            return pred

        g1, cg1, gw2, cg2, ggm, gbt = _mlp_parts(procs[i]['glob'])
        glob_args = (g0, g0 if first else g_cur, esums,
                     g1, cg1, gw2, cg2, ggm, gbt)
        table_args = (w1_l[i + 1], b1_l[i + 1], v1_l[i + 1], c1_l[i + 1])
        r = _pc_node_mlp(n0, n_cur, parts, cn, v1_l[i], v2_l[i], c2_l[i],
                         ngm_l[i], nbt_l[i], glob_args, table_args,
                         first, e_count)
        n_new, tA, tB, nsum, g_new, ce, cn = r

        ec_h = e_new_h
        n_cur, g_cur = n_new, g_new


# speedup vs baseline: 1.1031x; 1.1031x over previous
import functools

import jax
import jax.numpy as jnp
from jax import lax
from jax.experimental import pallas as pl
from jax.experimental.pallas import tpu as pltpu
from jax.experimental.pallas import tpu_sc as plsc

F32 = jnp.float32
BF16 = jnp.bfloat16

_NC = 2
_NS = 16
_NW = _NC * _NS

_BN = 2000
_BE = 1280
_CH = 200

_SPLIT = (81920, 78080)


def _dot(a, b):
    return lax.dot_general(a, b, (((1,), (0,)), ((), ())),
                           precision=lax.Precision.DEFAULT,
                           preferred_element_type=F32)


def _ln(h, g, b):
    d = h.shape[-1]
    s1 = jnp.sum(h, axis=-1, keepdims=True)
    s2 = jnp.sum(h * h, axis=-1, keepdims=True)
    mu = s1 * (1.0 / d)
    var = s2 * (1.0 / d) - mu * mu
    return (h - mu) * lax.rsqrt(var + 1e-5) * g + b


def _mlp2_ln(pre1, w2, b2, g, b):
    h = jnp.maximum(pre1, 0.0)
    h = jnp.maximum(_dot(h, w2) + b2, 0.0)
    return _ln(h, g, b)


def _row(v):
    return v.reshape(1, -1)


def _mlp_parts(p):
    (l1, l2) = p['layers']
    return (l1['w'], _row(l1['b']), l2['w'], _row(l2['b']),
            _row(p['norm']['g']), _row(p['norm']['b']))


def _full(shape):
    return pl.BlockSpec(shape, lambda i: tuple(0 for _ in shape))


def _rows(b, d, off=0):
    return pl.BlockSpec((b, d), lambda i: (i + off, 0))


def _pc_encode_edge(ea, p):
    e, din = ea.shape
    ea_t = ea.T
    be = 1280
    w1, b1, w2, b2, g, b = _mlp_parts(p)

    def kfn(ea_r, w1_r, b1_r, w2_r, b2_r, g_r, b_r, o_r):
        pre = lax.dot_general(ea_r[...], w1_r[...], (((0,), (0,)), ((), ())),
                              precision=lax.Precision.DEFAULT,
                              preferred_element_type=F32) + b1_r[...]
        o_r[...] = _mlp2_ln(pre, w2_r[...], b2_r[...], g_r[...],
                            b_r[...]).astype(BF16)

    return pl.pallas_call(
        kfn,
        grid=(e // be,),
        in_specs=[pl.BlockSpec((din, be), lambda i: (0, i)),
                  _full(w1.shape), _full(b1.shape),
                  _full(w2.shape), _full(b2.shape), _full(g.shape), _full(b.shape)],
        out_specs=_rows(be, 128),
        out_shape=jax.ShapeDtypeStruct((e, 128), BF16),
    )(ea_t, w1, b1, w2, b2, g, b)


def _pc_encode_node(x, u, pn, pg, w1n, b1n, v1n, c1n):
    n, din = x.shape
    w1, b1, w2, b2, g, b = _mlp_parts(pn)
    uw1, ub1, uw2, ub2, ug, ub = _mlp_parts(pg)

    def kfn(x_r, w1_r, b1_r, w2_r, b2_r, g_r, b_r,
            u_r, uw1_r, ub1_r, uw2_r, ub2_r, ug_r, ub_r,
            w1n_r, b1n_r, v1n_r, c1n_r,
            n_o, g_o, a_o, b_o, ce_o, cn_o):
        pre = _dot(x_r[...], w1_r[...]) + b1_r[...]
        n0 = _mlp2_ln(pre, w2_r[...], b2_r[...], g_r[...], b_r[...])
        n_o[...] = n0
        a_o[...] = _dot(n0, w1n_r[256:384, :] + w1n_r[384:512, :])
        b_o[...] = _dot(n0, w1n_r[512:640, :] + w1n_r[640:768, :])

        @pl.when(pl.program_id(0) == 0)
        def _():
            gp = _dot(u_r[...], uw1_r[...]) + ub1_r[...]
            g0 = _mlp2_ln(gp, uw2_r[...], ub2_r[...], ug_r[...], ub_r[...])
            g_o[...] = g0
            ce_o[...] = _dot(g0, w1n_r[768:896, :] + w1n_r[896:1024, :]) + b1n_r[...]
            cn_o[...] = _dot(g0, v1n_r[384:512, :] + v1n_r[512:640, :]) + c1n_r[...]

    return pl.pallas_call(
        kfn,
        grid=(n // _BN,),
        in_specs=[_rows(_BN, din)] + [_full(a.shape) for a in
                                      (w1, b1, w2, b2, g, b,
                                       u, uw1, ub1, uw2, ub2, ug, ub,
                                       w1n, b1n, v1n, c1n)],
        out_specs=(_rows(_BN, 128), _full((1, 128)),
                   _rows(_BN, 128), _rows(_BN, 128),
                   _full((1, 128)), _full((1, 128))),
        out_shape=tuple(jax.ShapeDtypeStruct(s, F32)
                        for s in ((n, 128), (1, 128), (n, 128), (n, 128),
                                  (1, 128), (1, 128))),
    )(x, w1, b1, w2, b2, g, b, u, uw1, ub1, uw2, ub2, ug, ub, w1n, b1n, v1n, c1n)


def _sc_gather(table_a, table_b, src_h, dst_h):
    e = src_h.shape[0]
    per_w = e // _NW
    n_full = per_w // _CH
    tail = per_w % _CH
    mesh = plsc.VectorSubcoreMesh(core_axis_name="c", subcore_axis_name="s")

    scratch = [pltpu.VMEM((_CH,), jnp.int32), pltpu.VMEM((_CH,), jnp.int32),
               pltpu.VMEM((_CH, 128), F32), pltpu.VMEM((_CH, 128), F32),
               pltpu.SemaphoreType.DMA, pltpu.SemaphoreType.DMA]
    if tail:
        scratch += [pltpu.VMEM((tail,), jnp.int32), pltpu.VMEM((tail,), jnp.int32),
                    pltpu.VMEM((tail, 128), F32), pltpu.VMEM((tail, 128), F32)]

    @functools.partial(
        pl.kernel,
        out_type=(jax.ShapeDtypeStruct((e, 128), F32),
                  jax.ShapeDtypeStruct((e, 128), F32)),
        mesh=mesh,
        scratch_types=scratch,
    )
    def k(ta, tb, s_h, d_h, ga, gb, ia, ib, ra, rb, sa, sb, *tails):
        wid = lax.axis_index("s") * _NC + lax.axis_index("c")
        base = wid * per_w

        def chunk(off, cia, cib, cra, crb, cn):
            pltpu.sync_copy(s_h.at[pl.ds(off, cn)], cia)
            pltpu.sync_copy(d_h.at[pl.ds(off, cn)], cib)
            cpa = pltpu.async_copy(ta.at[cia], cra, sa)
            cpb = pltpu.async_copy(tb.at[cib], crb, sb)
            cpa.wait()
            cpb.wait()
            pltpu.sync_copy(cra, ga.at[pl.ds(off, cn)])
            pltpu.sync_copy(crb, gb.at[pl.ds(off, cn)])

        def body(ci, carry):
            chunk(base + ci * _CH, ia, ib, ra, rb, _CH)
            return carry

        lax.fori_loop(0, n_full, body, 0)
        if tail:
            ia_t, ib_t, ra_t, rb_t = tails
            chunk(base + n_full * _CH, ia_t, ib_t, ra_t, rb_t, tail)

    return k(table_a, table_b, src_h, dst_h)


def _sc_scatter(e2_h, dst_h, zeros_nd):
    e = dst_h.shape[0]
    n = zeros_nd.shape[0]
    per_w = e // _NW
    n_full = per_w // _CH
    tail = per_w % _CH
    rpt = (n // _NS) // 8 * 8
    rtail = n - _NS * rpt
    mesh = plsc.VectorSubcoreMesh(core_axis_name="c", subcore_axis_name="s")

    scratch = [pltpu.VMEM((_CH,), jnp.int32), pltpu.VMEM((_CH, 128), F32),
               pltpu.VMEM_SHARED((n, 128), F32)]
    if tail:
        scratch += [pltpu.VMEM((tail,), jnp.int32), pltpu.VMEM((tail, 128), F32)]

    @functools.partial(
        pl.kernel,
        out_type=jax.ShapeDtypeStruct((2, n, 128), F32),
        mesh=mesh,
        scratch_types=scratch,
    )
    def k(e2h, d_h, z_h, out_h, idx, buf, acc, *tails):
        cid = lax.axis_index("c")
        sid = lax.axis_index("s")
        wid = sid * _NC + cid

        @pl.when(sid == 0)
        def _():
            pltpu.sync_copy(z_h, acc)

        plsc.subcore_barrier()
        base = wid * per_w

        def chunk(off, cidx, cbuf, cn):
            pltpu.sync_copy(d_h.at[pl.ds(off, cn)], cidx)
            pltpu.sync_copy(e2h.at[pl.ds(off, cn)], cbuf)
            pltpu.sync_copy(cbuf, acc.at[cidx], add=True)

        def body(ci, carry):
            chunk(base + ci * _CH, idx, buf, _CH)
            return carry

        lax.fori_loop(0, n_full, body, 0)
        if tail:
            idx_t, buf_t = tails
            chunk(base + n_full * _CH, idx_t, buf_t, tail)
        plsc.subcore_barrier()
        r0 = sid * rpt
        pltpu.sync_copy(acc.at[pl.ds(r0, rpt)], out_h.at[cid, pl.ds(r0, rpt)])
        if rtail:
            @pl.when(sid == _NS - 1)
            def _():
                t0 = _NS * rpt
                pltpu.sync_copy(acc.at[pl.ds(t0, rtail)],
                                out_h.at[cid, pl.ds(t0, rtail)])

    return k(e2_h, dst_h, zeros_nd)


def _pc_edge_mlp(e0, ec, ga, gb, w1, ce, w2, b2, g, b, want_sum, off_r, first, be):
    e = ga.shape[0]
    off_b = off_r // be

    def kfn(*refs):
        if first:
            (e0_r, ga_r, gb_r, w1_r, ce_r, w2_r, b2_r, g_r, b_r) = refs[:9]
            outs = refs[9:]
            pre = (_dot(e0_r[...].astype(F32), w1_r[0:128, :] + w1_r[128:256, :]) +
                   ga_r[...] + gb_r[...] + ce_r[...])
        else:
            (e0_r, ec_r, ga_r, gb_r, w1_r, ce_r, w2_r, b2_r, g_r, b_r) = refs[:10]
            outs = refs[10:]
            pre = (_dot(e0_r[...].astype(F32), w1_r[0:128, :]) +
                   _dot(ec_r[...], w1_r[128:256, :]) +
                   ga_r[...] + gb_r[...] + ce_r[...])
        e2 = _mlp2_ln(pre, w2_r[...], b2_r[...], g_r[...], b_r[...])
        outs[0][...] = e2
        if want_sum:
            s = jnp.sum(e2, axis=0, keepdims=True)

            @pl.when(pl.program_id(0) == 0)
            def _():
                outs[1][...] = s

            @pl.when(pl.program_id(0) != 0)
            def _():
                outs[1][...] += s

    ins = ([e0] if first else [e0, ec]) + [ga, gb, w1, ce, w2, b2, g, b]
    in_specs = [_rows(be, 128, off=off_b)] + \
               [_rows(be, 128)] * (2 if first else 3) + \
               [_full(a.shape) for a in ins[(3 if first else 4):]]
    out_specs = (_rows(be, 128),) + ((_full((1, 128)),) if want_sum else ())
    out_shape = ((jax.ShapeDtypeStruct((e, 128), F32),) +
                 ((jax.ShapeDtypeStruct((1, 128), F32),) if want_sum else ()))
    r = pl.pallas_call(
        kfn, grid=(e // be,), in_specs=in_specs,
        out_specs=out_specs if want_sum else out_specs[0],
        out_shape=out_shape if want_sum else out_shape[0],
    )(*ins)
    return r if want_sum else (r, None)


def _pc_node_mlp(n0, nc, parts, cn, v1, v2, c2, g, b,
                 glob_args, table_args, first, e_count):
    n = n0.shape[0]
    do_glob = glob_args is not None
    do_tab = table_args is not None
    nb = n // _BN
    np_ = len(parts)

    if do_glob:
        (g0, gc, esums, g1, cg1, gw2, cg2, ggm, gbt) = glob_args
    if do_tab:
        (w1n, b1n, v1n, c1n) = table_args

    def kfn(*refs):
        i = 0
        n0_r = refs[i]; i += 1
        if not first:
            nc_r = refs[i]; i += 1
        p_rs = refs[i:i + np_]; i += np_
        cn_r, v1_r, v2_r, c2_r, g_r, b_r = refs[i:i + 6]; i += 6
        if do_glob:
            es_rs = refs[i:i + len(esums)]; i += len(esums)
            (g0_r, gc_r, g1_r, cg1_r, gw2_r, cg2_r, ggm_r, gbt_r) = refs[i:i + 8]
            i += 8
        if do_tab:
            (w1n_r, b1n_r, v1n_r, c1n_r) = refs[i:i + 4]; i += 4
        outs = refs[i:]

        agg = p_rs[0][0] + p_rs[0][1]
        for p_r in p_rs[1:]:
            agg = agg + p_r[0] + p_r[1]
        if first:
            pre = (_dot(n0_r[...], v1_r[0:128, :] + v1_r[128:256, :]) +
                   _dot(agg, v1_r[256:384, :]) + cn_r[...])
        else:
            pre = (_dot(n0_r[...], v1_r[0:128, :]) +
                   _dot(nc_r[...], v1_r[128:256, :]) +
                   _dot(agg, v1_r[256:384, :]) + cn_r[...])
        n2 = _mlp2_ln(pre, v2_r[...], c2_r[...], g_r[...], b_r[...])
        outs[0][...] = n2
        oi = 1
        if do_tab:
            outs[oi][...] = _dot(n0_r[...], w1n_r[256:384, :]) + \
                _dot(n2, w1n_r[384:512, :])
            outs[oi + 1][...] = _dot(n0_r[...], w1n_r[512:640, :]) + \
                _dot(n2, w1n_r[640:768, :])
            oi += 2
        if do_glob:
            s = jnp.sum(n2, axis=0, keepdims=True)

            @pl.when(pl.program_id(0) == 0)
            def _():
                outs[oi][...] = s

            @pl.when(pl.program_id(0) != 0)
            def _():
                outs[oi][...] += s

            @pl.when(pl.program_id(0) == nb - 1)
            def _():
                nm = outs[oi][...] * (1.0 / n)
                em = es_rs[0][...]
                for es_r in es_rs[1:]:
                    em = em + es_r[...]
                em = em * (1.0 / e_count)
                if first:
                    gpre = _dot(g0_r[...], g1_r[0:128, :] + g1_r[128:256, :])
                else:
                    gpre = (_dot(g0_r[...], g1_r[0:128, :]) +
                            _dot(gc_r[...], g1_r[128:256, :]))
                gpre += (_dot(nm, g1_r[256:384, :]) +
                         _dot(em, g1_r[384:512, :]) + cg1_r[...])
                g2 = _mlp2_ln(gpre, gw2_r[...], cg2_r[...], ggm_r[...], gbt_r[...])
                outs[oi + 1][...] = g2
                if do_tab:
                    outs[oi + 2][...] = (_dot(g0_r[...], w1n_r[768:896, :]) +
                                         _dot(g2, w1n_r[896:1024, :]) + b1n_r[...])
                    outs[oi + 3][...] = (_dot(g0_r[...], v1n_r[384:512, :]) +
                                         _dot(g2, v1n_r[512:640, :]) + c1n_r[...])

    ins = [n0] + ([] if first else [nc]) + list(parts) + [cn, v1, v2, c2, g, b]
    if do_glob:
        ins += list(esums) + [g0, gc, g1, cg1, gw2, cg2, ggm, gbt]
    if do_tab:
        ins += [w1n, b1n, v1n, c1n]
    n_row = 1 + (0 if first else 1)
    in_specs = ([_rows(_BN, 128)] * n_row +
                [pl.BlockSpec((2, _BN, 128), lambda i: (0, i, 0))] * np_ +
                [_full(a.shape) for a in ins[n_row + np_:]])

    out_specs = [_rows(_BN, 128)]
    out_shape = [jax.ShapeDtypeStruct((n, 128), F32)]
    if do_tab:
        out_specs += [_rows(_BN, 128), _rows(_BN, 128)]
        out_shape += [jax.ShapeDtypeStruct((n, 128), F32)] * 2
    if do_glob:
        out_specs += [_full((1, 128)), _full((1, 128))]
        out_shape += [jax.ShapeDtypeStruct((1, 128), F32)] * 2
        if do_tab:
            out_specs += [_full((1, 128)), _full((1, 128))]
            out_shape += [jax.ShapeDtypeStruct((1, 128), F32)] * 2
    r = pl.pallas_call(
        kfn, grid=(nb,), in_specs=in_specs,
        out_specs=tuple(out_specs), out_shape=tuple(out_shape),
    )(*ins)
    return r


def _pc_node_decode(n0, nc, parts, cn, v1, v2, c2, g, b, y, pd, po):
    n = n0.shape[0]
    nb = n // _BN
    np_ = len(parts)
    dw1, db1, dw2, db2, dg, dbt = _mlp_parts(pd)
    ow = po['layers'][0]['w']
    ob = po['layers'][0]['b']
    owp = jnp.zeros((128, 128), F32).at[:, :ow.shape[1]].set(ow)
    obp = jnp.zeros((1, 128), F32).at[0, :ob.shape[0]].set(ob)
    ny = y.shape[0]

    def kfn(*refs):
        (n0_r, nc_r) = refs[:2]
        p_rs = refs[2:2 + np_]
        (cn_r, v1_r, v2_r, c2_r, g_r, b_r,
         y_r, dw1_r, db1_r, dw2_r, db2_r, dg_r, dbt_r,
         owp_r, obp_r, acc) = refs[2 + np_:]

        agg = p_rs[0][0] + p_rs[0][1]
        for p_r in p_rs[1:]:
            agg = agg + p_r[0] + p_r[1]
        pre = (_dot(n0_r[...], v1_r[0:128, :]) +
               _dot(nc_r[...], v1_r[128:256, :]) +
               _dot(agg, v1_r[256:384, :]) + cn_r[...])
        n2 = _mlp2_ln(pre, v2_r[...], c2_r[...], g_r[...], b_r[...])

        base = _dot(n2, dw1_r[0:128, :]) + db1_r[...]
        yc = _dot(y_r[...], dw1_r[128:, :])

        @pl.when(pl.program_id(0) == 0)
        def _():
            acc[...] = jnp.full((ny, 128), jnp.inf, F32)

        for i in range(ny):
            h = _mlp2_ln(base + yc[i:i + 1, :], dw2_r[...], db2_r[...],
                         dg_r[...], dbt_r[...])
            o = _dot(h, owp_r[...]) + obp_r[...]
            m = jnp.min(o, axis=0, keepdims=True)
            acc[i:i + 1, :] = jnp.minimum(acc[i:i + 1, :], m)

    ins = [n0, nc] + list(parts) + [cn, v1, v2, c2, g, b,
                                    y, dw1, db1, dw2, db2, dg, dbt, owp, obp]
    in_specs = ([_rows(_BN, 128)] * 2 +
                [pl.BlockSpec((2, _BN, 128), lambda i: (0, i, 0))] * np_ +
                [_full(a.shape) for a in ins[2 + np_:]])
    acc = pl.pallas_call(
        kfn, grid=(nb,), in_specs=in_specs,
        out_specs=_full((ny, 128)),
        out_shape=jax.ShapeDtypeStruct((ny, 128), F32),
    )(*ins)
    return acc[:, :ow.shape[1]].reshape(-1)


def kernel(edge_attr, edge_index, x, y, z, u, batch, params):
    del z, batch
    e_count = edge_attr.shape[0]
    n_count = x.shape[0]

    if e_count == sum(_SPLIT):
        sizes = _SPLIT
    else:
        sizes = (e_count,)
    bounds = []
    o = 0
    for sz in sizes:
        bounds.append((o, sz))
        o += sz

    src_h = [lax.slice_in_dim(edge_index[0], s, s + sz) for s, sz in bounds]
    dst_h = [lax.slice_in_dim(edge_index[1], s, s + sz) for s, sz in bounds]

    enc = params['encoder']
    procs = params['processors']
    e0 = _pc_encode_edge(edge_attr, enc['edge'])

    w1_l, b1_l, w2_l, b2_l, egm_l, ebt_l = zip(*[_mlp_parts(p['edge'])
                                                 for p in procs])
    v1_l, c1_l, v2_l, c2_l, ngm_l, nbt_l = zip(*[_mlp_parts(p['node'])
                                                 for p in procs])

    n0, g0, tA, tB, ce, cn = _pc_encode_node(
        x, u, enc['node'], enc['glob'], w1_l[0], b1_l[0], v1_l[0], c1_l[0])

    zeros_nd = jnp.zeros((n_count, 128), F32)
    ec_h = [None] * len(bounds)
    n_cur = g_cur = None
    for i in range(3):
        first = i == 0
        last = i == 2

        gab_h = [_sc_gather(tA, tB, s, d) for s, d in zip(src_h, dst_h)]

        e_new_h = []
        esums = []
        parts = []
        for hi, (off, sz) in enumerate(bounds):
            ga, gb = gab_h[hi]
            e_new, esum = _pc_edge_mlp(
                e0, ec_h[hi], ga, gb, w1_l[i], ce, w2_l[i], b2_l[i],
                egm_l[i], ebt_l[i], want_sum=not last, off_r=off,
                first=first, be=2048 if off % 2048 == 0 and sz % 2048 == 0 else _BE)
            e_new_h.append(e_new)
            if esum is not None:
                esums.append(esum)
            parts.append(_sc_scatter(e_new, dst_h[hi], zeros_nd))

        if last:
            pred = _pc_node_decode(n0, n_cur, parts, cn, v1_l[i], v2_l[i],
                                   c2_l[i], ngm_l[i], nbt_l[i], y,
                                   params['decoder']['node'],
                                   params['output_transformer']['node'])
            return pred

        g1, cg1, gw2, cg2, ggm, gbt = _mlp_parts(procs[i]['glob'])
        glob_args = (g0, g0 if first else g_cur, esums,
                     g1, cg1, gw2, cg2, ggm, gbt)
        table_args = (w1_l[i + 1], b1_l[i + 1], v1_l[i + 1], c1_l[i + 1])
        r = _pc_node_mlp(n0, n_cur, parts, cn, v1_l[i], v2_l[i], c2_l[i],
                         ngm_l[i], nbt_l[i], glob_args, table_args,
                         first, e_count)
        n_new, tA, tB, nsum, g_new, ce, cn = r

        ec_h = e_new_h
        n_cur, g_cur = n_new, g_new
